# baseline (speedup 1.0000x reference)
import functools

import jax
import jax.numpy as jnp
from jax import lax
from jax.experimental import pallas as pl
from jax.experimental.pallas import tpu as pltpu
from jax.experimental.pallas import tpu_sc as plsc

D = 64
L = 16


def kernel(user_ids, item_ids, user_table, item_table, fc_w, fc_b):
    batch = user_ids.shape[0]
    info = plsc.get_sparse_core_info()
    nw = info.num_cores * info.num_subcores
    bpw = batch // nw
    n_groups = bpw // L

    wb = jnp.concatenate(
        [fc_w[:, 0], jnp.full((L,), fc_b[0], jnp.float32)]
    )

    mesh = plsc.VectorSubcoreMesh(core_axis_name="c", subcore_axis_name="s")

    @functools.partial(
        pl.kernel,
        mesh=mesh,
        out_type=jax.ShapeDtypeStruct((batch,), jnp.float32),
        compiler_params=pltpu.CompilerParams(
            needs_layout_passes=False, use_tc_tiling_on_sc=False
        ),
        scratch_types=[
            pltpu.VMEM((bpw,), jnp.int32),
            pltpu.VMEM((bpw,), jnp.int32),
            pltpu.VMEM((bpw, D), jnp.float32),
            pltpu.VMEM((bpw, D), jnp.float32),
            pltpu.VMEM((2 * D + L,), jnp.float32),
            pltpu.VMEM((L, bpw + 1), jnp.float32),
            pltpu.VMEM((bpw,), jnp.float32),
            pltpu.SemaphoreType.DMA,
            pltpu.SemaphoreType.DMA,
        ],
    )
    def sc_kernel(uid_hbm, iid_hbm, ut_hbm, it_hbm, wb_hbm, out_hbm,
                  uidx_v, iidx_v, urows_v, irows_v, wb_v, part_v, out_v,
                  sem_u, sem_i):
        wid = lax.axis_index("s") * info.num_cores + lax.axis_index("c")
        base = wid * bpw
        pltpu.sync_copy(wb_hbm, wb_v)
        pltpu.sync_copy(uid_hbm.at[pl.ds(base, bpw)], uidx_v)
        pltpu.sync_copy(iid_hbm.at[pl.ds(base, bpw)], iidx_v)
        cu = pltpu.async_copy(ut_hbm.at[uidx_v], urows_v, sem_u)
        ci = pltpu.async_copy(it_hbm.at[iidx_v], irows_v, sem_i)
        cu.wait()
        ci.wait()

        wu = [wb_v[pl.ds(k * L, L)] for k in range(D // L)]
        wi = [wb_v[pl.ds(D + k * L, L)] for k in range(D // L)]
        bias_vec = wb_v[pl.ds(2 * D, L)]
        lane = lax.iota(jnp.int32, L)

        def row_body(b, carry):
            acc = urows_v[b, pl.ds(0, L)] * wu[0]
            for k in range(1, D // L):
                acc = acc + urows_v[b, pl.ds(k * L, L)] * wu[k]
            for k in range(D // L):
                acc = acc + irows_v[b, pl.ds(k * L, L)] * wi[k]
            col = jnp.full((L,), b, jnp.int32)
            plsc.store_scatter(part_v, [lane, col], acc)
            return carry

        lax.fori_loop(0, bpw, row_body, 0)

        def group_body(g, carry):
            s = bias_vec
            for l in range(L):
                s = s + part_v[l, pl.ds(g * L, L)]
            out_v[pl.ds(pl.multiple_of(g * L, L), L)] = s
            return carry

        lax.fori_loop(0, n_groups, group_body, 0)
        pltpu.sync_copy(out_v, out_hbm.at[pl.ds(base, bpw)])

    return sc_kernel(user_ids, item_ids, user_table, item_table, wb)


# speedup vs baseline: 5.2831x; 5.2831x over previous
import functools

import jax
import jax.numpy as jnp
from jax import lax
from jax.experimental import pallas as pl
from jax.experimental.pallas import tpu as pltpu
from jax.experimental.pallas import tpu_sc as plsc

D = 64
L = 16
CB = 8192


def _matvec_body(ut_ref, it_ref, wu_ref, wi_ref, pu_ref, pi_ref):
    pu_ref[...] = jnp.sum(ut_ref[...] * wu_ref[...], axis=0)
    pi_ref[...] = jnp.sum(it_ref[...] * wi_ref[...], axis=0)


def _score_vectors(ut_t, it_t, wu, wi):
    n = ut_t.shape[1]
    grid = (n + CB - 1) // CB
    return pl.pallas_call(
        _matvec_body,
        grid=(grid,),
        in_specs=[
            pl.BlockSpec((D, CB), lambda i: (0, i)),
            pl.BlockSpec((D, CB), lambda i: (0, i)),
            pl.BlockSpec((D, 1), lambda i: (0, 0)),
            pl.BlockSpec((D, 1), lambda i: (0, 0)),
        ],
        out_specs=[
            pl.BlockSpec((CB,), lambda i: (i,)),
            pl.BlockSpec((CB,), lambda i: (i,)),
        ],
        out_shape=[
            jax.ShapeDtypeStruct((n,), jnp.float32),
            jax.ShapeDtypeStruct((n,), jnp.float32),
        ],
    )(ut_t, it_t, wu, wi)


def kernel(user_ids, item_ids, user_table, item_table, fc_w, fc_b):
    batch = user_ids.shape[0]
    info = plsc.get_sparse_core_info()
    nw = info.num_cores * info.num_subcores
    bpw = batch // nw

    ut_t = user_table.T
    it_t = item_table.T
    wu = fc_w[:D]
    wi = fc_w[D:]

    pu, pi = _score_vectors(ut_t, it_t, wu, wi)
    pu = pu + fc_b[0]

    mesh = plsc.VectorSubcoreMesh(core_axis_name="c", subcore_axis_name="s")

    @functools.partial(
        pl.kernel,
        mesh=mesh,
        out_type=jax.ShapeDtypeStruct((batch,), jnp.float32),
        compiler_params=pltpu.CompilerParams(
            needs_layout_passes=False, use_tc_tiling_on_sc=False
        ),
        scratch_types=[
            pltpu.VMEM((bpw,), jnp.int32),
            pltpu.VMEM((bpw,), jnp.int32),
            pltpu.VMEM((bpw,), jnp.float32),
            pltpu.VMEM((bpw,), jnp.float32),
            pltpu.VMEM((bpw,), jnp.float32),
            pltpu.SemaphoreType.DMA,
            pltpu.SemaphoreType.DMA,
        ],
    )
    def sc_gather(uid_hbm, iid_hbm, pu_hbm, pi_hbm, out_hbm,
                  uidx_v, iidx_v, puv_v, piv_v, out_v, sem_u, sem_i):
        wid = lax.axis_index("s") * info.num_cores + lax.axis_index("c")
        base = wid * bpw
        pltpu.sync_copy(uid_hbm.at[pl.ds(base, bpw)], uidx_v)
        pltpu.sync_copy(iid_hbm.at[pl.ds(base, bpw)], iidx_v)
        cu = pltpu.async_copy(pu_hbm.at[uidx_v], puv_v, sem_u)
        ci = pltpu.async_copy(pi_hbm.at[iidx_v], piv_v, sem_i)
        cu.wait()
        ci.wait()
        for g in range(bpw // L):
            out_v[pl.ds(g * L, L)] = (
                puv_v[pl.ds(g * L, L)] + piv_v[pl.ds(g * L, L)]
            )
        pltpu.sync_copy(out_v, out_hbm.at[pl.ds(base, bpw)])

    return sc_gather(user_ids, item_ids, pu, pi)


# speedup vs baseline: 5.3691x; 1.0163x over previous
import functools

import jax
import jax.numpy as jnp
from jax import lax
from jax.experimental import pallas as pl
from jax.experimental.pallas import tpu as pltpu
from jax.experimental.pallas import tpu_sc as plsc

D = 64
L = 16
CB = 8192


def _matvec_body(b_ref, ut_ref, it_ref, wu_ref, wi_ref, pu_ref, pi_ref):
    pu_ref[...] = jnp.sum(ut_ref[...] * wu_ref[...], axis=0) + b_ref[0]
    pi_ref[...] = jnp.sum(it_ref[...] * wi_ref[...], axis=0)


def _score_vectors(ut_t, it_t, wu, wi, fc_b):
    n = ut_t.shape[1]
    grid = (n + CB - 1) // CB
    return pl.pallas_call(
        _matvec_body,
        grid=(grid,),
        in_specs=[
            pl.BlockSpec(memory_space=pltpu.SMEM),
            pl.BlockSpec((D, CB), lambda i: (0, i)),
            pl.BlockSpec((D, CB), lambda i: (0, i)),
            pl.BlockSpec((D, 1), lambda i: (0, 0)),
            pl.BlockSpec((D, 1), lambda i: (0, 0)),
        ],
        out_specs=[
            pl.BlockSpec((CB,), lambda i: (i,)),
            pl.BlockSpec((CB,), lambda i: (i,)),
        ],
        out_shape=[
            jax.ShapeDtypeStruct((n,), jnp.float32),
            jax.ShapeDtypeStruct((n,), jnp.float32),
        ],
    )(fc_b, ut_t, it_t, wu, wi)


def kernel(user_ids, item_ids, user_table, item_table, fc_w, fc_b):
    batch = user_ids.shape[0]
    info = plsc.get_sparse_core_info()
    nw = info.num_cores * info.num_subcores
    bpw = batch // nw

    ut_t = user_table.T
    it_t = item_table.T
    wu = fc_w[:D]
    wi = fc_w[D:]

    pu, pi = _score_vectors(ut_t, it_t, wu, wi, fc_b)

    mesh = plsc.VectorSubcoreMesh(core_axis_name="c", subcore_axis_name="s")

    @functools.partial(
        pl.kernel,
        mesh=mesh,
        out_type=jax.ShapeDtypeStruct((batch,), jnp.float32),
        compiler_params=pltpu.CompilerParams(
            needs_layout_passes=False, use_tc_tiling_on_sc=False
        ),
        scratch_types=[
            pltpu.VMEM((bpw,), jnp.int32),
            pltpu.VMEM((bpw,), jnp.int32),
            pltpu.VMEM((bpw,), jnp.float32),
            pltpu.VMEM((bpw,), jnp.float32),
            pltpu.VMEM((bpw,), jnp.float32),
            pltpu.SemaphoreType.DMA,
            pltpu.SemaphoreType.DMA,
        ],
    )
    def sc_gather(uid_hbm, iid_hbm, pu_hbm, pi_hbm, out_hbm,
                  uidx_v, iidx_v, puv_v, piv_v, out_v, sem_u, sem_i):
        wid = lax.axis_index("s") * info.num_cores + lax.axis_index("c")
        base = wid * bpw
        pltpu.sync_copy(uid_hbm.at[pl.ds(base, bpw)], uidx_v)
        pltpu.sync_copy(iid_hbm.at[pl.ds(base, bpw)], iidx_v)
        cu = pltpu.async_copy(pu_hbm.at[uidx_v], puv_v, sem_u)
        ci = pltpu.async_copy(pi_hbm.at[iidx_v], piv_v, sem_i)
        cu.wait()
        ci.wait()
        for g in range(bpw // L):
            out_v[pl.ds(g * L, L)] = (
                puv_v[pl.ds(g * L, L)] + piv_v[pl.ds(g * L, L)]
            )
        pltpu.sync_copy(out_v, out_hbm.at[pl.ds(base, bpw)])

    return sc_gather(user_ids, item_ids, pu, pi)


# speedup vs baseline: 6.2582x; 1.1656x over previous
import functools

import jax
import jax.numpy as jnp
from jax import lax
from jax.experimental import pallas as pl
from jax.experimental.pallas import tpu as pltpu
from jax.experimental.pallas import tpu_sc as plsc

D = 64
L = 16
CB = 16384


def _matvec_body(b_ref, ut_ref, it_ref, wu_ref, wi_ref, pu_ref, pi_ref):
    pu_ref[...] = jnp.sum(ut_ref[...] * wu_ref[...], axis=0) + b_ref[0]
    pi_ref[...] = jnp.sum(it_ref[...] * wi_ref[...], axis=0)


def _score_vectors(ut_t, it_t, wu, wi, fc_b):
    n = ut_t.shape[1]
    grid = (n + CB - 1) // CB
    return pl.pallas_call(
        _matvec_body,
        grid=(grid,),
        in_specs=[
            pl.BlockSpec(memory_space=pltpu.SMEM),
            pl.BlockSpec((D, CB), lambda i: (0, i)),
            pl.BlockSpec((D, CB), lambda i: (0, i)),
            pl.BlockSpec((D, 1), lambda i: (0, 0)),
            pl.BlockSpec((D, 1), lambda i: (0, 0)),
        ],
        out_specs=[
            pl.BlockSpec((CB,), lambda i: (i,)),
            pl.BlockSpec((CB,), lambda i: (i,)),
        ],
        out_shape=[
            jax.ShapeDtypeStruct((n,), jnp.float32),
            jax.ShapeDtypeStruct((n,), jnp.float32),
        ],
    )(fc_b, ut_t, it_t, wu, wi)


def kernel(user_ids, item_ids, user_table, item_table, fc_w, fc_b):
    batch = user_ids.shape[0]
    info = plsc.get_sparse_core_info()
    nw = info.num_cores * info.num_subcores
    bpw = batch // nw

    ut_t = user_table.T
    it_t = item_table.T
    wu = fc_w[:D]
    wi = fc_w[D:]

    pu, pi = _score_vectors(ut_t, it_t, wu, wi, fc_b)

    mesh = plsc.VectorSubcoreMesh(core_axis_name="c", subcore_axis_name="s")

    @functools.partial(
        pl.kernel,
        mesh=mesh,
        out_type=jax.ShapeDtypeStruct((batch,), jnp.float32),
        compiler_params=pltpu.CompilerParams(
            needs_layout_passes=False, use_tc_tiling_on_sc=False
        ),
        scratch_types=[
            pltpu.VMEM((bpw,), jnp.int32),
            pltpu.VMEM((bpw,), jnp.int32),
            pltpu.VMEM((bpw,), jnp.float32),
            pltpu.VMEM((bpw,), jnp.float32),
            pltpu.VMEM((bpw,), jnp.float32),
            pltpu.SemaphoreType.DMA,
            pltpu.SemaphoreType.DMA,
        ],
    )
    def sc_gather(uid_hbm, iid_hbm, pu_hbm, pi_hbm, out_hbm,
                  uidx_v, iidx_v, puv_v, piv_v, out_v, sem_u, sem_i):
        wid = lax.axis_index("s") * info.num_cores + lax.axis_index("c")
        base = wid * bpw
        pltpu.sync_copy(uid_hbm.at[pl.ds(base, bpw)], uidx_v)
        pltpu.sync_copy(iid_hbm.at[pl.ds(base, bpw)], iidx_v)
        cu = pltpu.async_copy(pu_hbm.at[uidx_v], puv_v, sem_u)
        ci = pltpu.async_copy(pi_hbm.at[iidx_v], piv_v, sem_i)
        cu.wait()
        ci.wait()
        for g in range(bpw // L):
            out_v[pl.ds(g * L, L)] = (
                puv_v[pl.ds(g * L, L)] + piv_v[pl.ds(g * L, L)]
            )
        pltpu.sync_copy(out_v, out_hbm.at[pl.ds(base, bpw)])

    return sc_gather(user_ids, item_ids, pu, pi)


# speedup vs baseline: 6.2669x; 1.0014x over previous
import functools

import jax
import jax.numpy as jnp
from jax import lax
from jax.experimental import pallas as pl
from jax.experimental.pallas import tpu as pltpu
from jax.experimental.pallas import tpu_sc as plsc

D = 64
L = 16
CB = 32768


def _matvec_body(b_ref, ut_ref, it_ref, wu_ref, wi_ref, pu_ref, pi_ref):
    pu_ref[...] = jnp.sum(ut_ref[...] * wu_ref[...], axis=0) + b_ref[0]
    pi_ref[...] = jnp.sum(it_ref[...] * wi_ref[...], axis=0)


def _score_vectors(ut_t, it_t, wu, wi, fc_b):
    n = ut_t.shape[1]
    grid = (n + CB - 1) // CB
    return pl.pallas_call(
        _matvec_body,
        grid=(grid,),
        in_specs=[
            pl.BlockSpec(memory_space=pltpu.SMEM),
            pl.BlockSpec((D, CB), lambda i: (0, i)),
            pl.BlockSpec((D, CB), lambda i: (0, i)),
            pl.BlockSpec((D, 1), lambda i: (0, 0)),
            pl.BlockSpec((D, 1), lambda i: (0, 0)),
        ],
        out_specs=[
            pl.BlockSpec((CB,), lambda i: (i,)),
            pl.BlockSpec((CB,), lambda i: (i,)),
        ],
        out_shape=[
            jax.ShapeDtypeStruct((n,), jnp.float32),
            jax.ShapeDtypeStruct((n,), jnp.float32),
        ],
    )(fc_b, ut_t, it_t, wu, wi)


def kernel(user_ids, item_ids, user_table, item_table, fc_w, fc_b):
    batch = user_ids.shape[0]
    info = plsc.get_sparse_core_info()
    nw = info.num_cores * info.num_subcores
    bpw = batch // nw

    ut_t = user_table.T
    it_t = item_table.T
    wu = fc_w[:D]
    wi = fc_w[D:]

    pu, pi = _score_vectors(ut_t, it_t, wu, wi, fc_b)

    mesh = plsc.VectorSubcoreMesh(core_axis_name="c", subcore_axis_name="s")

    @functools.partial(
        pl.kernel,
        mesh=mesh,
        out_type=jax.ShapeDtypeStruct((batch,), jnp.float32),
        compiler_params=pltpu.CompilerParams(
            needs_layout_passes=False, use_tc_tiling_on_sc=False
        ),
        scratch_types=[
            pltpu.VMEM((bpw,), jnp.int32),
            pltpu.VMEM((bpw,), jnp.int32),
            pltpu.VMEM((bpw,), jnp.float32),
            pltpu.VMEM((bpw,), jnp.float32),
            pltpu.VMEM((bpw,), jnp.float32),
            pltpu.SemaphoreType.DMA,
            pltpu.SemaphoreType.DMA,
        ],
    )
    def sc_gather(uid_hbm, iid_hbm, pu_hbm, pi_hbm, out_hbm,
                  uidx_v, iidx_v, puv_v, piv_v, out_v, sem_u, sem_i):
        wid = lax.axis_index("s") * info.num_cores + lax.axis_index("c")
        base = wid * bpw
        pltpu.sync_copy(uid_hbm.at[pl.ds(base, bpw)], uidx_v)
        pltpu.sync_copy(iid_hbm.at[pl.ds(base, bpw)], iidx_v)
        cu = pltpu.async_copy(pu_hbm.at[uidx_v], puv_v, sem_u)
        ci = pltpu.async_copy(pi_hbm.at[iidx_v], piv_v, sem_i)
        cu.wait()
        ci.wait()
        for g in range(bpw // L):
            out_v[pl.ds(g * L, L)] = (
                puv_v[pl.ds(g * L, L)] + piv_v[pl.ds(g * L, L)]
            )
        pltpu.sync_copy(out_v, out_hbm.at[pl.ds(base, bpw)])

    return sc_gather(user_ids, item_ids, pu, pi)
